# Initial kernel scaffold; baseline (speedup 1.0000x reference)
import jax
import jax.numpy as jnp
from jax.experimental import pallas as pl


def kernel(x, edge_index, Wl0, bl0, Wr0, Wl1, bl1, Wr1):
    raise NotImplementedError("write your pallas kernel here")


# speedup vs baseline: 1.2165x; 1.2165x over previous
import functools

import jax
import jax.numpy as jnp
from jax import lax
from jax.experimental import pallas as pl
from jax.experimental.pallas import tpu as pltpu
from jax.experimental.pallas import tpu_sc as plsc

N = 10000
E = 320000
D = 128
NC = 2
NS = 16
NW = NC * NS
EPW = E // NW
OWN = 320
NP = NW * OWN
PAD_ROW = OWN
SCN = 400
RB = 128
RBS = RB + 16
FH = 64
SUBCAP = EPW + FH
DEGW = 16
_f32 = jnp.float32
_i32 = jnp.int32


def _compact_body(src_hbm, dst_hbm, srcl_hbm, dstl_hbm, cnt_hbm,
                  es_v, ed_v, ring_s, ring_d, cnt_v, cur_smem):
    cid = lax.axis_index("c")
    sid = lax.axis_index("s")
    wid = sid * NC + cid

    def _zc(j, carry):
        cur_smem[j] = jnp.int32(0)
        return carry
    lax.fori_loop(0, NW, _zc, None)

    def _flush(o, c_end):
        half = c_end // FH - 1
        boff = (half - (half // 2) * 2) * FH
        base = (o * NW + wid) * SUBCAP + half * FH
        pltpu.sync_copy(ring_s.at[pl.ds(o * RBS + boff, FH)],
                        srcl_hbm.at[pl.ds(base, FH)])
        pltpu.sync_copy(ring_d.at[pl.ds(o * RBS + boff, FH)],
                        dstl_hbm.at[pl.ds(base, FH)])

    def _scan(i, carry):
        e0 = wid * EPW + i * SCN
        pltpu.sync_copy(src_hbm.at[pl.ds(e0, SCN)], es_v)
        pltpu.sync_copy(dst_hbm.at[pl.ds(e0, SCN)], ed_v)

        def _vreg(g, carry):
            src16 = es_v[pl.ds(g * 16, 16)]
            dst16 = ed_v[pl.ds(g * 16, 16)]
            for lane in range(16):
                d = dst16[lane]
                s = src16[lane]
                o = d // OWN
                dl = d - o * OWN
                c = cur_smem[o]
                cl = c & (RB - 1)
                ring_s[pl.ds(o * RBS + cl, 16)] = jnp.full((16,), s, _i32)
                ring_d[pl.ds(o * RBS + cl, 16)] = jnp.full((16,), dl, _i32)
                c2 = c + 1
                cur_smem[o] = c2

                @pl.when((c2 & (FH - 1)) == 0)
                def _():
                    _flush(o, c2)
            return carry
        return lax.fori_loop(0, SCN // 16, _vreg, carry)
    lax.fori_loop(0, EPW // SCN, _scan, None)

    zeros16 = jnp.zeros((16,), _i32)
    pads16 = jnp.full((16,), PAD_ROW, _i32)

    def _drain(o, carry):
        c = cur_smem[o]
        cpad = ((c + FH - 1) // FH) * FH

        @pl.when(cpad > c)
        def _():
            def _pad(j, carry2):
                cl = (c & (RB - 1)) + j * 16

                @pl.when(cl < RB)
                def _():
                    ring_s[pl.ds(o * RBS + cl, 16)] = zeros16
                    ring_d[pl.ds(o * RBS + cl, 16)] = pads16

                @pl.when(cl >= RB)
                def _():
                    ring_s[pl.ds(o * RBS + cl - RB, 16)] = zeros16
                    ring_d[pl.ds(o * RBS + cl - RB, 16)] = pads16
                return carry2
            lax.fori_loop(0, (cpad - c + 15) // 16, _pad, None)
            _flush(o, cpad)
        cnt_v[pl.ds(o * 16, 16)] = jnp.full((16,), cpad // FH, _i32)
        return carry
    lax.fori_loop(0, NW, _drain, None)
    pltpu.sync_copy(cnt_v, cnt_hbm.at[pl.ds(wid * NW * 16, NW * 16)])


def _make_compact():
    mesh = plsc.VectorSubcoreMesh(core_axis_name="c", subcore_axis_name="s",
                                  num_cores=NC, num_subcores=NS)
    return pl.kernel(
        _compact_body,
        out_type=(jax.ShapeDtypeStruct((NW * NW * SUBCAP,), _i32),
                  jax.ShapeDtypeStruct((NW * NW * SUBCAP,), _i32),
                  jax.ShapeDtypeStruct((NW * NW * 16,), _i32)),
        mesh=mesh,
        scratch_types=[
            pltpu.VMEM((SCN,), _i32),
            pltpu.VMEM((SCN,), _i32),
            pltpu.VMEM((NW * RBS,), _i32),
            pltpu.VMEM((NW * RBS,), _i32),
            pltpu.VMEM((NW * 16,), _i32),
            pltpu.SMEM((NW,), _i32),
        ],
        name="sage_sc_compact",
    )


def _agg_body(compute_deg, h_hbm, srcl_hbm, dstl_hbm, cnt_hbm, *refs):
    if compute_deg:
        agg_hbm, deg_hbm, src_v, dstl_v, rows_v, acc_v, deg_v, cnt_v, sem = refs
    else:
        agg_hbm, src_v, dstl_v, rows_v, acc_v, cnt_v, sem = refs
    cid = lax.axis_index("c")
    sid = lax.axis_index("s")
    wid = sid * NC + cid

    zf = jnp.zeros((16,), _f32)

    def _za(i, carry):
        acc_v[i // (D // 16), pl.ds((i % (D // 16)) * 16, 16)] = zf
        return carry
    lax.fori_loop(0, (OWN + 1) * (D // 16), _za, None)
    if compute_deg:
        def _zd(i, carry):
            deg_v[i, pl.ds(0, 16)] = zf
            return carry
        lax.fori_loop(0, OWN + 1, _zd, None)
        one0 = jnp.where(lax.iota(_i32, 16) == 0, 1.0, 0.0).astype(_f32)

    def _bucket(j, carry):
        pltpu.sync_copy(cnt_hbm.at[pl.ds((j * NW + wid) * 16, 16)], cnt_v)
        trips = cnt_v[pl.ds(0, 16)][0]

        def _chunk(i, carry2):
            off = (wid * NW + j) * SUBCAP + i * FH
            pltpu.sync_copy(srcl_hbm.at[pl.ds(off, FH)], src_v)
            pltpu.sync_copy(dstl_hbm.at[pl.ds(off, FH)], dstl_v)
            pltpu.async_copy(h_hbm.at[src_v], rows_v, sem).wait()

            def _grp(g, cc):
                dl16 = dstl_v[pl.ds(g * 16, 16)]
                for lane in range(16):
                    dl = dl16[lane]
                    for cblk in range(D // 16):
                        sl = pl.ds(cblk * 16, 16)
                        acc_v[dl, sl] = (acc_v[dl, sl]
                                         + rows_v[g * 16 + lane, sl])
                    if compute_deg:
                        dsl = pl.ds(0, 16)
                        deg_v[dl, dsl] = deg_v[dl, dsl] + one0
                return cc
            lax.fori_loop(0, FH // 16, _grp, None)
            return carry2
        lax.fori_loop(0, trips, _chunk, None)
        return carry
    lax.fori_loop(0, NW, _bucket, None)

    pltpu.sync_copy(acc_v.at[pl.ds(0, OWN)], agg_hbm.at[pl.ds(wid * OWN, OWN)])
    if compute_deg:
        pltpu.sync_copy(deg_v.at[pl.ds(0, OWN)],
                        deg_hbm.at[pl.ds(wid * OWN, OWN)])


def _make_agg(compute_deg):
    out_type = [jax.ShapeDtypeStruct((NP, D), _f32)]
    scratch = [
        pltpu.VMEM((FH,), _i32),
        pltpu.VMEM((FH,), _i32),
        pltpu.VMEM((FH, D), _f32),
        pltpu.VMEM((OWN + 1, D), _f32),
    ]
    if compute_deg:
        out_type.append(jax.ShapeDtypeStruct((NP, DEGW), _f32))
        scratch.append(pltpu.VMEM((OWN + 1, DEGW), _f32))
    scratch.append(pltpu.VMEM((16,), _i32))
    scratch.append(pltpu.SemaphoreType.DMA)
    mesh = plsc.VectorSubcoreMesh(core_axis_name="c", subcore_axis_name="s",
                                  num_cores=NC, num_subcores=NS)
    return pl.kernel(
        functools.partial(_agg_body, compute_deg),
        out_type=tuple(out_type) if compute_deg else out_type[0],
        mesh=mesh,
        scratch_types=scratch,
        name="sage_sc_agg",
    )


def _tc_body(agg_ref, deg_ref, h_ref, wl_ref, bl_ref, wr_ref, o_ref):
    agg = agg_ref[...]
    deg = deg_ref[...][:, 0:1]
    mean = agg / jnp.maximum(deg, 1.0)
    dn = (((1,), (1,)), ((), ()))
    out = lax.dot_general(mean, wl_ref[...], dn,
                          preferred_element_type=_f32,
                          precision=lax.Precision.HIGHEST)
    out = out + lax.dot_general(h_ref[...], wr_ref[...], dn,
                                preferred_element_type=_f32,
                                precision=lax.Precision.HIGHEST)
    out = out + bl_ref[...]
    nrm = jnp.sqrt(jnp.sum(out * out, axis=-1, keepdims=True))
    out = out / jnp.maximum(nrm, 1e-12)
    o_ref[...] = jnp.maximum(out, 0.0)


def _tc_layer(agg, deg, h, Wl, bl, Wr):
    blk = 1000
    grid = (N // blk,)
    return pl.pallas_call(
        _tc_body,
        grid=grid,
        in_specs=[
            pl.BlockSpec((blk, D), lambda i: (i, 0)),
            pl.BlockSpec((blk, DEGW), lambda i: (i, 0)),
            pl.BlockSpec((blk, D), lambda i: (i, 0)),
            pl.BlockSpec((D, D), lambda i: (0, 0)),
            pl.BlockSpec((1, D), lambda i: (0, 0)),
            pl.BlockSpec((D, D), lambda i: (0, 0)),
        ],
        out_specs=pl.BlockSpec((blk, D), lambda i: (i, 0)),
        out_shape=jax.ShapeDtypeStruct((N, D), _f32),
        name="sage_tc_layer",
    )(agg, deg, h, Wl, bl.reshape(1, D), Wr)


def kernel(x, edge_index, Wl0, bl0, Wr0, Wl1, bl1, Wr1):
    src = edge_index[0]
    dst = edge_index[1]
    srcl, dstl, cnts = _make_compact()(src, dst)
    agg0, deg = _make_agg(True)(x, srcl, dstl, cnts)
    h1 = _tc_layer(agg0, deg, x, Wl0, bl0, Wr0)
    agg1 = _make_agg(False)(h1, srcl, dstl, cnts)
    h2 = _tc_layer(agg1, deg, h1, Wl1, bl1, Wr1)
    return h2
